# baseline (speedup 1.0000x reference)
import functools

import jax
import jax.numpy as jnp
from jax import lax
from jax.experimental import pallas as pl
from jax.experimental.pallas import tpu as pltpu
from jax.experimental.pallas import tpu_sc as plsc

NC = 2
NS = 16
NW = NC * NS
L = 16

N = 128 * 32768
E = N // NW
C = 8192
NCH = E // C
NB1 = 16384
NB2 = 2048
UNROLL = 8


def _histo_body(is_pass2, a_hbm, b_hbm, m_hbm, bsel_hbm, out_hbm,
                av0, av1, bv0, bv1, mv0, mv1, bselv, accv, hc, sa, sb, sm):
    abufs, bbufs, mbufs = (av0, av1), (bv0, bv1), (mv0, mv1)
    wid = lax.axis_index("s") * NC + lax.axis_index("c")
    base = wid * E
    zeros = jnp.zeros((L,), jnp.float32)
    ones = jnp.ones((L,), jnp.float32)

    nb = NB2 if is_pass2 else NB1

    def zero_body(j, _):
        hc[pl.ds(j * L, L)] = zeros
        return _
    lax.fori_loop(0, nb // L, zero_body, None, unroll=8)

    if is_pass2:
        pltpu.sync_copy(bsel_hbm, bselv)
        bsel = bselv[...]

    def start(c, slot):
        off = base + c * C
        return (pltpu.async_copy(a_hbm.at[pl.ds(off, C)], abufs[slot], sa.at[slot]),
                pltpu.async_copy(b_hbm.at[pl.ds(off, C)], bbufs[slot], sb.at[slot]),
                pltpu.async_copy(m_hbm.at[pl.ds(off, C)], mbufs[slot], sm.at[slot]))

    acc = zeros
    pending = start(0, 0)
    for c in range(NCH):
        slot = c & 1
        nxt = start(c + 1, slot ^ 1) if c + 1 < NCH else None
        for h in pending:
            h.wait()
        pending = nxt
        avs, bvs, mvs = abufs[slot], bbufs[slot], mbufs[slot]

        def inner(i, acc):
            s = i * L
            a = avs[pl.ds(s, L)]
            b = bvs[pl.ds(s, L)]
            m = mvs[pl.ds(s, L)]
            sel = m != 0
            d = jnp.abs(a - b)
            u = lax.bitcast_convert_type(d, jnp.int32)
            idx1 = lax.shift_right_logical(u, 17)
            if is_pass2:
                idx = lax.bitwise_and(lax.shift_right_logical(u, 6),
                                      jnp.int32(NB2 - 1))
                inb = sel & (idx1 == bsel)
                acc = acc + jnp.where(sel & (idx1 > bsel), d, jnp.float32(0.0))
            else:
                idx = idx1
                inb = sel
            plsc.addupdate_scatter(hc, [idx], ones, mask=inb)
            return acc
        acc = lax.fori_loop(0, C // L, inner, acc, unroll=UNROLL)

    accv[...] = acc
    pltpu.sync_copy(hc, out_hbm.at[wid, pl.ds(0, nb)])
    pltpu.sync_copy(accv, out_hbm.at[wid, pl.ds(nb, L)])


def _make_pass(is_pass2):
    mesh = plsc.VectorSubcoreMesh(core_axis_name="c", subcore_axis_name="s",
                                  num_cores=NC, num_subcores=NS)
    nb = NB2 if is_pass2 else NB1
    return pl.kernel(
        functools.partial(_histo_body, is_pass2),
        out_type=jax.ShapeDtypeStruct((NW, nb + L), jnp.float32),
        mesh=mesh,
        scratch_types=[
            pltpu.VMEM((C,), jnp.float32),
            pltpu.VMEM((C,), jnp.float32),
            pltpu.VMEM((C,), jnp.float32),
            pltpu.VMEM((C,), jnp.float32),
            pltpu.VMEM((C,), jnp.int32),
            pltpu.VMEM((C,), jnp.int32),
            pltpu.VMEM((L,), jnp.int32),
            pltpu.VMEM((L,), jnp.float32),
            pltpu.VMEM((nb,), jnp.float32),
            pltpu.SemaphoreType.DMA((2,)),
            pltpu.SemaphoreType.DMA((2,)),
            pltpu.SemaphoreType.DMA((2,)),
        ],
        compiler_params=pltpu.CompilerParams(needs_layout_passes=False),
        name="ohem_histo2" if is_pass2 else "ohem_histo1",
    )


def _select_bin(cnt, k, nb):
    cum = jnp.cumsum(cnt[::-1])[::-1]
    b = jnp.clip(jnp.sum((cum >= k).astype(jnp.int32)) - 1, 0, nb - 1)
    cump = jnp.concatenate([cum, jnp.zeros((1,), cum.dtype)])
    return b, cump[b + 1]


def kernel(inputs, targets, mask):
    a = inputs.reshape(-1)
    b = targets.reshape(-1)
    m = mask.reshape(-1).astype(jnp.int32)

    pass1 = _make_pass(False)
    pass2 = _make_pass(True)

    h1 = pass1(a, b, m, jnp.zeros((L,), jnp.int32))
    cnt1 = jnp.sum(h1[:, :NB1], axis=0).astype(jnp.int32)
    num_selected = jnp.sum(cnt1)
    k = (num_selected * 6) // 10
    b1, cnt_gt1 = _select_bin(cnt1, k, NB1)

    h2 = pass2(a, b, m, jnp.full((L,), b1, jnp.int32))
    cnt2f = jnp.sum(h2[:, :NB2], axis=0)
    cnt2 = cnt2f.astype(jnp.int32)
    sum_hi = jnp.sum(h2[:, NB2:])

    k2 = k - cnt_gt1
    b2, cnt_gt2 = _select_bin(cnt2, k2, NB2)

    j = jnp.arange(NB2, dtype=jnp.int32)
    mids = lax.bitcast_convert_type((b1 << 17) | (j << 6) | 32, jnp.float32)
    sum_mid = jnp.sum(jnp.where(j > b2, cnt2f * mids, 0.0))
    t_mid = lax.bitcast_convert_type((b1 << 17) | (b2 << 6) | 31, jnp.float32)
    rem = (k2 - cnt_gt2).astype(jnp.float32)
    total = sum_hi + sum_mid + rem * t_mid
    return total / k.astype(jnp.float32)


# speedup vs baseline: 1.2149x; 1.2149x over previous
import functools

import jax
import jax.numpy as jnp
from jax import lax
from jax.experimental import pallas as pl
from jax.experimental.pallas import tpu as pltpu
from jax.experimental.pallas import tpu_sc as plsc

NC = 2
NS = 16
NW = NC * NS
L = 16

R, Q = 128, 32768
N = R * Q
E = N // NW
C = 16384
NCH = E // C
NB1 = 16384
NB2 = 2048
UNROLL = 8


def _prep_body(a_ref, b_ref, m_ref, o_ref):
    bits = lax.bitcast_convert_type(a_ref[...] - b_ref[...], jnp.int32)
    bits = lax.bitwise_and(bits, jnp.int32(0x7FFFFFFF))
    o_ref[...] = jnp.where(m_ref[...], bits, jnp.int32(-1))


_prep = pl.pallas_call(
    _prep_body,
    grid=(16,),
    in_specs=[pl.BlockSpec((8, Q), lambda i: (i, 0))] * 3,
    out_specs=pl.BlockSpec((8, Q), lambda i: (i, 0)),
    out_shape=jax.ShapeDtypeStruct((R, Q), jnp.int32),
)


def _histo_body(is_pass2, w_hbm, bsel_hbm, out_hbm,
                wv0, wv1, bselv, accv, hc, sw):
    wbufs = (wv0, wv1)
    wid = lax.axis_index("s") * NC + lax.axis_index("c")
    base = wid * E
    zeros = jnp.zeros((L,), jnp.float32)
    ones = jnp.ones((L,), jnp.float32)
    nb = NB2 if is_pass2 else NB1

    def zero_body(j, _):
        hc[pl.ds(j * L, L)] = zeros
        return _
    lax.fori_loop(0, nb // L, zero_body, None, unroll=8)

    if is_pass2:
        pltpu.sync_copy(bsel_hbm, bselv)
        bsel = bselv[...]

    def start(c, slot):
        off = base + c * C
        return pltpu.async_copy(w_hbm.at[pl.ds(off, C)], wbufs[slot],
                                sw.at[slot])

    acc = zeros
    pending = start(0, 0)
    for c in range(NCH):
        slot = c & 1
        nxt = start(c + 1, slot ^ 1) if c + 1 < NCH else None
        pending.wait()
        pending = nxt
        wvs = wbufs[slot]

        def inner(i, acc):
            w = wvs[pl.ds(i * L, L)]
            sel = w >= 0
            idx1 = lax.shift_right_logical(w, 17)
            if is_pass2:
                idx = lax.bitwise_and(lax.shift_right_logical(w, 6),
                                      jnp.int32(NB2 - 1))
                inb = sel & (idx1 == bsel)
                d = lax.bitcast_convert_type(w, jnp.float32)
                acc = acc + jnp.where(sel & (idx1 > bsel), d, jnp.float32(0.0))
            else:
                idx = idx1
                inb = sel
            plsc.addupdate_scatter(hc, [idx], ones, mask=inb)
            return acc
        acc = lax.fori_loop(0, C // L, inner, acc, unroll=UNROLL)

    accv[...] = acc
    pltpu.sync_copy(hc, out_hbm.at[wid, pl.ds(0, nb)])
    pltpu.sync_copy(accv, out_hbm.at[wid, pl.ds(nb, L)])


def _make_pass(is_pass2):
    mesh = plsc.VectorSubcoreMesh(core_axis_name="c", subcore_axis_name="s",
                                  num_cores=NC, num_subcores=NS)
    nb = NB2 if is_pass2 else NB1
    return pl.kernel(
        functools.partial(_histo_body, is_pass2),
        out_type=jax.ShapeDtypeStruct((NW, nb + L), jnp.float32),
        mesh=mesh,
        scratch_types=[
            pltpu.VMEM((C,), jnp.int32),
            pltpu.VMEM((C,), jnp.int32),
            pltpu.VMEM((L,), jnp.int32),
            pltpu.VMEM((L,), jnp.float32),
            pltpu.VMEM((nb,), jnp.float32),
            pltpu.SemaphoreType.DMA((2,)),
        ],
        compiler_params=pltpu.CompilerParams(needs_layout_passes=False),
        name="ohem_histo2" if is_pass2 else "ohem_histo1",
    )


def _select_bin(cnt, k, nb):
    cum = jnp.cumsum(cnt[::-1])[::-1]
    b = jnp.clip(jnp.sum((cum >= k).astype(jnp.int32)) - 1, 0, nb - 1)
    cump = jnp.concatenate([cum, jnp.zeros((1,), cum.dtype)])
    return b, cump[b + 1]


def kernel(inputs, targets, mask):
    w = _prep(inputs, targets, mask).reshape(-1)

    pass1 = _make_pass(False)
    pass2 = _make_pass(True)

    h1 = pass1(w, jnp.zeros((L,), jnp.int32))
    cnt1 = jnp.sum(h1[:, :NB1], axis=0).astype(jnp.int32)
    num_selected = jnp.sum(cnt1)
    k = (num_selected * 6) // 10
    b1, cnt_gt1 = _select_bin(cnt1, k, NB1)

    h2 = pass2(w, jnp.full((L,), b1, jnp.int32))
    cnt2f = jnp.sum(h2[:, :NB2], axis=0)
    cnt2 = cnt2f.astype(jnp.int32)
    sum_hi = jnp.sum(h2[:, NB2:])

    k2 = k - cnt_gt1
    b2, cnt_gt2 = _select_bin(cnt2, k2, NB2)

    j = jnp.arange(NB2, dtype=jnp.int32)
    mids = lax.bitcast_convert_type((b1 << 17) | (j << 6) | 32, jnp.float32)
    sum_mid = jnp.sum(jnp.where(j > b2, cnt2f * mids, 0.0))
    t_mid = lax.bitcast_convert_type((b1 << 17) | (b2 << 6) | 31, jnp.float32)
    rem = (k2 - cnt_gt2).astype(jnp.float32)
    total = sum_hi + sum_mid + rem * t_mid
    return total / k.astype(jnp.float32)


# speedup vs baseline: 1.7737x; 1.4600x over previous
import jax
import jax.numpy as jnp
from jax import lax
from jax.experimental import pallas as pl
from jax.experimental.pallas import tpu as pltpu
from jax.experimental.pallas import tpu_sc as plsc

NC = 2
NS = 16
NW = NC * NS
L = 16

R, Q = 128, 32768
N = R * Q
E = N // NW
C = 16384
NCH = E // C
NB = 65536
SH = 15
UNROLL = 8


def _prep_body(a_ref, b_ref, m_ref, o_ref):
    bits = lax.bitcast_convert_type(a_ref[...] - b_ref[...], jnp.int32)
    bits = lax.bitwise_and(bits, jnp.int32(0x7FFFFFFF))
    o_ref[...] = jnp.where(m_ref[...], bits, jnp.int32(-1))


_prep = pl.pallas_call(
    _prep_body,
    grid=(16,),
    in_specs=[pl.BlockSpec((8, Q), lambda i: (i, 0))] * 3,
    out_specs=pl.BlockSpec((8, Q), lambda i: (i, 0)),
    out_shape=jax.ShapeDtypeStruct((R, Q), jnp.int32),
)


def _histo_body(w_hbm, out_hbm, wv0, wv1, hc, sw):
    wbufs = (wv0, wv1)
    wid = lax.axis_index("s") * NC + lax.axis_index("c")
    base = wid * E
    zeros = jnp.zeros((L,), jnp.float32)
    ones = jnp.ones((L,), jnp.float32)

    def zero_body(j, _):
        hc[pl.ds(j * L, L)] = zeros
        return _
    lax.fori_loop(0, NB // L, zero_body, None, unroll=8)

    def start(c, slot):
        off = base + c * C
        return pltpu.async_copy(w_hbm.at[pl.ds(off, C)], wbufs[slot],
                                sw.at[slot])

    pending = start(0, 0)
    for c in range(NCH):
        slot = c & 1
        nxt = start(c + 1, slot ^ 1) if c + 1 < NCH else None
        pending.wait()
        pending = nxt
        wvs = wbufs[slot]

        def inner(i, _):
            w = wvs[pl.ds(i * L, L)]
            plsc.addupdate_scatter(hc, [lax.shift_right_logical(w, SH)],
                                   ones, mask=w >= 0)
            return _
        lax.fori_loop(0, C // L, inner, None, unroll=UNROLL)

    pltpu.sync_copy(hc, out_hbm.at[wid])


_mesh = plsc.VectorSubcoreMesh(core_axis_name="c", subcore_axis_name="s",
                               num_cores=NC, num_subcores=NS)
_histo = pl.kernel(
    _histo_body,
    out_type=jax.ShapeDtypeStruct((NW, NB), jnp.float32),
    mesh=_mesh,
    scratch_types=[
        pltpu.VMEM((C,), jnp.int32),
        pltpu.VMEM((C,), jnp.int32),
        pltpu.VMEM((NB,), jnp.float32),
        pltpu.SemaphoreType.DMA((2,)),
    ],
    compiler_params=pltpu.CompilerParams(needs_layout_passes=False),
    name="ohem_histo",
)


def kernel(inputs, targets, mask):
    w = _prep(inputs, targets, mask).reshape(-1)
    h = _histo(w)

    cntf = jnp.sum(h, axis=0)
    cnt = cntf.astype(jnp.int32)
    num_selected = jnp.sum(cnt)
    k = (num_selected * 6) // 10

    cum = jnp.cumsum(cnt[::-1])[::-1]
    b = jnp.clip(jnp.sum((cum >= k).astype(jnp.int32)) - 1, 0, NB - 1)
    cump = jnp.concatenate([cum, jnp.zeros((1,), cum.dtype)])
    cnt_gt = cump[b + 1]

    j = jnp.arange(NB, dtype=jnp.int32)
    mids = lax.bitcast_convert_type((j << SH) | (1 << (SH - 1)), jnp.float32)
    sum_mid = jnp.sum(jnp.where(j > b, cntf * mids, 0.0))
    rem = (k - cnt_gt).astype(jnp.float32)
    total = sum_mid + rem * mids[b]
    return total / k.astype(jnp.float32)


# speedup vs baseline: 1.7765x; 1.0016x over previous
import jax
import jax.numpy as jnp
from jax import lax
from jax.experimental import pallas as pl
from jax.experimental.pallas import tpu as pltpu
from jax.experimental.pallas import tpu_sc as plsc

NC = 2
NS = 16
NW = NC * NS
L = 16

R, Q = 128, 32768
N = R * Q
E = N // NW
C = 16384
NCH = E // C
NB = 65536
SH = 15
UNROLL = 8


def _prep_body(a_ref, b_ref, m_ref, o_ref):
    bits = lax.bitcast_convert_type(a_ref[...] - b_ref[...], jnp.int32)
    bits = lax.bitwise_and(bits, jnp.int32(0x7FFFFFFF))
    o_ref[...] = jnp.where(m_ref[...], bits, jnp.int32(-1))


_prep = pl.pallas_call(
    _prep_body,
    grid=(16,),
    in_specs=[pl.BlockSpec((8, Q), lambda i: (i, 0))] * 3,
    out_specs=pl.BlockSpec((8, Q), lambda i: (i, 0)),
    out_shape=jax.ShapeDtypeStruct((R, Q), jnp.int32),
)


def _histo_body(w_hbm, out_hbm, wv0, wv1, hc, sw):
    wbufs = (wv0, wv1)
    wid = lax.axis_index("s") * NC + lax.axis_index("c")
    base = wid * E
    zeros = jnp.zeros((L,), jnp.float32)
    ones = jnp.ones((L,), jnp.float32)

    def zero_body(j, _):
        hc[pl.ds(j * L, L)] = zeros
        return _
    lax.fori_loop(0, NB // L, zero_body, None, unroll=8)

    def start(c, slot):
        off = base + c * C
        return pltpu.async_copy(w_hbm.at[pl.ds(off, C)], wbufs[slot],
                                sw.at[slot])

    pending = start(0, 0)
    for c in range(NCH):
        slot = c & 1
        nxt = start(c + 1, slot ^ 1) if c + 1 < NCH else None
        pending.wait()
        pending = nxt
        wvs = wbufs[slot]

        def inner(i, _):
            w = wvs[pl.ds(i * L, L)]
            plsc.addupdate_scatter(hc, [lax.shift_right_logical(w, SH)],
                                   ones, mask=w >= 0)
            return _
        lax.fori_loop(0, C // L, inner, None, unroll=UNROLL)

    pltpu.sync_copy(hc, out_hbm.at[wid])


_mesh = plsc.VectorSubcoreMesh(core_axis_name="c", subcore_axis_name="s",
                               num_cores=NC, num_subcores=NS)
_histo = pl.kernel(
    _histo_body,
    out_type=jax.ShapeDtypeStruct((NW, NB), jnp.float32),
    mesh=_mesh,
    scratch_types=[
        pltpu.VMEM((C,), jnp.int32),
        pltpu.VMEM((C,), jnp.int32),
        pltpu.VMEM((NB,), jnp.float32),
        pltpu.SemaphoreType.DMA((2,)),
    ],
    compiler_params=pltpu.CompilerParams(needs_layout_passes=False),
    name="ohem_histo",
)


def kernel(inputs, targets, mask):
    w = _prep(inputs, targets, mask).reshape(-1)
    h = _histo(w)

    cntf = jnp.sum(h, axis=0)
    cnt = cntf.astype(jnp.int32)
    num_selected = jnp.sum(cnt)
    k = (num_selected * 6) // 10

    cum = jnp.cumsum(cnt[::-1])[::-1]
    b = jnp.clip(jnp.sum((cum >= k).astype(jnp.int32)) - 1, 0, NB - 1)
    cump = jnp.concatenate([cum, jnp.zeros((1,), cum.dtype)])
    cnt_gt = cump[b + 1]

    j = jnp.arange(NB, dtype=jnp.int32)
    mids = lax.bitcast_convert_type((j << SH) | (1 << (SH - 1)), jnp.float32)
    sum_mid = jnp.sum(jnp.where((j > b) & (cntf > 0), cntf * mids, 0.0))
    rem = (k - cnt_gt).astype(jnp.float32)
    total = sum_mid + rem * mids[b]
    return total / k.astype(jnp.float32)
